# baseline (speedup 1.0000x reference)
import functools

import jax
import jax.numpy as jnp
from jax import lax
from jax.experimental import pallas as pl
from jax.experimental.pallas import tpu as pltpu
from jax.experimental.pallas import tpu_sc as plsc

NC = 2
NS = 16
L = 16
RND = 64

_PARAMS = pltpu.CompilerParams(
    needs_layout_passes=False, use_tc_tiling_on_sc=False)


def _mesh():
    return plsc.VectorSubcoreMesh(core_axis_name="c", subcore_axis_name="s")


def _fetch_rows(tab_hbm, raw, rows, rbase, sem):
    copies = []
    for gg in range(RND // L):
        v_vec = raw[pl.ds(rbase + gg * L, L)]
        for k in range(L):
            copies.append(pltpu.async_copy(
                tab_hbm.at[pl.ds(v_vec[k], 1)],
                rows.at[pl.ds(gg * L + k, 1)], sem))
    return copies


@functools.cache
def _make_k1(V: int, D: int, B: int):
    NW = NC * NS
    bpw = B // NW
    n_rounds = bpw // RND

    @functools.partial(
        pl.kernel,
        out_type=jax.ShapeDtypeStruct((B, D), jnp.float32),
        mesh=_mesh(),
        compiler_params=_PARAMS,
        scratch_types=[
            pltpu.VMEM((bpw,), jnp.int32),
            pltpu.VMEM((RND, D), jnp.float32),
            pltpu.SemaphoreType.DMA,
        ],
    )
    def k1(i_hbm, wi_hbm, out_hbm, raw_i, rows_i, sem):
        wid = lax.axis_index("s") * NC + lax.axis_index("c")
        base = wid * bpw
        pltpu.sync_copy(i_hbm.at[pl.ds(base, bpw)], raw_i)

        def round_body(r, carry):
            rbase = r * RND
            for cp in _fetch_rows(wi_hbm, raw_i, rows_i, rbase, sem):
                cp.wait()
            pltpu.sync_copy(
                rows_i, out_hbm.at[pl.ds(base + rbase, RND)])
            return carry

        lax.fori_loop(0, n_rounds, round_body, 0)

    return k1


@functools.cache
def _make_k2(V: int, D: int, B: int):
    NW = NC * NS
    bpw = B // NW
    n_rounds = bpw // RND

    @functools.partial(
        pl.kernel,
        out_type=jax.ShapeDtypeStruct((B,), jnp.float32),
        mesh=_mesh(),
        compiler_params=_PARAMS,
        scratch_types=[
            pltpu.VMEM((bpw,), jnp.int32),
            pltpu.VMEM((bpw,), jnp.int32),
            pltpu.VMEM((bpw,), jnp.int32),
            pltpu.VMEM((bpw,), jnp.int32),
            pltpu.VMEM((RND, D), jnp.float32),
            pltpu.VMEM((RND, D), jnp.float32),
            pltpu.VMEM((RND, 128), jnp.float32),
            pltpu.VMEM((RND, 128), jnp.float32),
            pltpu.VMEM((bpw,), jnp.float32),
            pltpu.SemaphoreType.DMA,
        ],
    )
    def k2(i_hbm, j_hbm, wirows_hbm, wj_hbm, bi_hbm, bj_hbm, out_hbm,
           raw_i, raw_j, bidx_i, bidx_j, rows_i, rows_j,
           brow_i, brow_j, out_v, sem):
        wid = lax.axis_index("s") * NC + lax.axis_index("c")
        base = wid * bpw
        pltpu.sync_copy(i_hbm.at[pl.ds(base, bpw)], raw_i)
        pltpu.sync_copy(j_hbm.at[pl.ds(base, bpw)], raw_j)
        for t in range(bpw // L):
            tsl = pl.ds(t * L, L)
            bidx_i[tsl] = jnp.right_shift(raw_i[tsl], 7)
            bidx_j[tsl] = jnp.right_shift(raw_j[tsl], 7)

        lane = lax.iota(jnp.int32, L)

        def round_body(r, carry):
            rbase = r * RND
            copies = _fetch_rows(wj_hbm, raw_j, rows_j, rbase, sem)
            copies.append(pltpu.async_copy(
                wirows_hbm.at[pl.ds(base + rbase, RND)], rows_i, sem))
            copies.append(pltpu.async_copy(
                bi_hbm.at[bidx_i.at[pl.ds(rbase, RND)]], brow_i, sem))
            copies.append(pltpu.async_copy(
                bj_hbm.at[bidx_j.at[pl.ds(rbase, RND)]], brow_j, sem))
            for cp in copies:
                cp.wait()

            for gg in range(RND // L):
                rr = rbase + gg * L
                tsl = pl.ds(rr, L)
                cvec = lane + gg * L
                acc = [jnp.zeros((L,), jnp.float32) for _ in range(4)]
                for d in range(D):
                    dvec = jnp.full((L,), d, jnp.int32)
                    acc[d % 4] = acc[d % 4] + (
                        plsc.load_gather(rows_i, [cvec, dvec])
                        * plsc.load_gather(rows_j, [cvec, dvec]))
                tot = (acc[0] + acc[1]) + (acc[2] + acc[3])
                mod_i = jnp.bitwise_and(raw_i[tsl], 127)
                mod_j = jnp.bitwise_and(raw_j[tsl], 127)
                tot = tot + plsc.load_gather(brow_i, [cvec, mod_i])
                tot = tot + plsc.load_gather(brow_j, [cvec, mod_j])
                out_v[tsl] = tot
            return carry

        lax.fori_loop(0, n_rounds, round_body, 0)
        pltpu.sync_copy(out_v, out_hbm.at[pl.ds(base, bpw)])

    return k2


def kernel(i_indices, j_indices, wi, wj, bi, bj):
    V, D = wi.shape
    B = i_indices.shape[0]
    vpad = (-V) % 128
    bi2 = jnp.pad(bi.T, ((0, 0), (0, vpad))).reshape(-1, 128)
    bj2 = jnp.pad(bj.T, ((0, 0), (0, vpad))).reshape(-1, 128)
    wirows = _make_k1(V, D, B)(i_indices, wi)
    return _make_k2(V, D, B)(i_indices, j_indices, wirows, wj, bi2, bj2)


# speedup vs baseline: 1.4888x; 1.4888x over previous
import functools

import jax
import jax.numpy as jnp
from jax import lax
from jax.experimental import pallas as pl
from jax.experimental.pallas import tpu as pltpu
from jax.experimental.pallas import tpu_sc as plsc

NC = 2
NS = 16
L = 16
RND = 64

_PARAMS = pltpu.CompilerParams(
    needs_layout_passes=False, use_tc_tiling_on_sc=True)


def _mesh():
    return plsc.VectorSubcoreMesh(core_axis_name="c", subcore_axis_name="s")


def _fetch_rows(tab_hbm, raw, rows, rbase, sem):
    copies = []
    for gg in range(RND // L):
        v_vec = raw[pl.ds(rbase + gg * L, L)]
        for k in range(L):
            copies.append(pltpu.async_copy(
                tab_hbm.at[pl.ds(v_vec[k], 1)],
                rows.at[pl.ds(gg * L + k, 1)], sem))
    return copies


@functools.cache
def _make_k1(V: int, D: int, B: int):
    NW = NC * NS
    bpw = B // NW
    n_rounds = bpw // RND

    @functools.partial(
        pl.kernel,
        out_type=jax.ShapeDtypeStruct((B, D), jnp.float32),
        mesh=_mesh(),
        compiler_params=_PARAMS,
        scratch_types=[
            pltpu.VMEM((bpw,), jnp.int32),
            pltpu.VMEM((RND, D), jnp.float32),
            pltpu.SemaphoreType.DMA,
        ],
    )
    def k1(i_hbm, wi_hbm, out_hbm, raw_i, rows_i, sem):
        wid = lax.axis_index("s") * NC + lax.axis_index("c")
        base = wid * bpw
        pltpu.sync_copy(i_hbm.at[pl.ds(base, bpw)], raw_i)

        def round_body(r, carry):
            rbase = r * RND
            for cp in _fetch_rows(wi_hbm, raw_i, rows_i, rbase, sem):
                cp.wait()
            pltpu.sync_copy(
                rows_i, out_hbm.at[pl.ds(base + rbase, RND)])
            return carry

        lax.fori_loop(0, n_rounds, round_body, 0)

    return k1


@functools.cache
def _make_k2(V: int, D: int, B: int):
    NW = NC * NS
    bpw = B // NW
    n_rounds = bpw // RND

    @functools.partial(
        pl.kernel,
        out_type=jax.ShapeDtypeStruct((B,), jnp.float32),
        mesh=_mesh(),
        compiler_params=_PARAMS,
        scratch_types=[
            pltpu.VMEM((bpw,), jnp.int32),
            pltpu.VMEM((bpw,), jnp.int32),
            pltpu.VMEM((bpw,), jnp.int32),
            pltpu.VMEM((bpw,), jnp.int32),
            pltpu.VMEM((RND, D), jnp.float32),
            pltpu.VMEM((RND, D), jnp.float32),
            pltpu.VMEM((RND, 128), jnp.float32),
            pltpu.VMEM((RND, 128), jnp.float32),
            pltpu.VMEM((bpw,), jnp.float32),
            pltpu.SemaphoreType.DMA,
        ],
    )
    def k2(i_hbm, j_hbm, wirows_hbm, wj_hbm, bi_hbm, bj_hbm, out_hbm,
           raw_i, raw_j, bidx_i, bidx_j, rows_i, rows_j,
           brow_i, brow_j, out_v, sem):
        wid = lax.axis_index("s") * NC + lax.axis_index("c")
        base = wid * bpw
        pltpu.sync_copy(i_hbm.at[pl.ds(base, bpw)], raw_i)
        pltpu.sync_copy(j_hbm.at[pl.ds(base, bpw)], raw_j)
        for t in range(bpw // L):
            tsl = pl.ds(t * L, L)
            bidx_i[tsl] = jnp.right_shift(raw_i[tsl], 7)
            bidx_j[tsl] = jnp.right_shift(raw_j[tsl], 7)

        lane = lax.iota(jnp.int32, L)

        def round_body(r, carry):
            rbase = r * RND
            copies = _fetch_rows(wj_hbm, raw_j, rows_j, rbase, sem)
            copies.append(pltpu.async_copy(
                wirows_hbm.at[pl.ds(base + rbase, RND)], rows_i, sem))
            copies.append(pltpu.async_copy(
                bi_hbm.at[bidx_i.at[pl.ds(rbase, RND)]], brow_i, sem))
            copies.append(pltpu.async_copy(
                bj_hbm.at[bidx_j.at[pl.ds(rbase, RND)]], brow_j, sem))
            for cp in copies:
                cp.wait()

            for gg in range(RND // L):
                rr = rbase + gg * L
                tsl = pl.ds(rr, L)
                cvec = lane + gg * L
                acc = [jnp.zeros((L,), jnp.float32) for _ in range(4)]
                for d in range(D):
                    dvec = jnp.full((L,), d, jnp.int32)
                    acc[d % 4] = acc[d % 4] + (
                        plsc.load_gather(rows_i, [cvec, dvec])
                        * plsc.load_gather(rows_j, [cvec, dvec]))
                tot = (acc[0] + acc[1]) + (acc[2] + acc[3])
                mod_i = jnp.bitwise_and(raw_i[tsl], 127)
                mod_j = jnp.bitwise_and(raw_j[tsl], 127)
                tot = tot + plsc.load_gather(brow_i, [cvec, mod_i])
                tot = tot + plsc.load_gather(brow_j, [cvec, mod_j])
                out_v[tsl] = tot
            return carry

        lax.fori_loop(0, n_rounds, round_body, 0)
        pltpu.sync_copy(out_v, out_hbm.at[pl.ds(base, bpw)])

    return k2


def kernel(i_indices, j_indices, wi, wj, bi, bj):
    V, D = wi.shape
    B = i_indices.shape[0]
    vpad = (-V) % 128
    bi2 = jnp.pad(bi.T, ((0, 0), (0, vpad))).reshape(-1, 128)
    bj2 = jnp.pad(bj.T, ((0, 0), (0, vpad))).reshape(-1, 128)
    wirows = _make_k1(V, D, B)(i_indices, wi)
    return _make_k2(V, D, B)(i_indices, j_indices, wirows, wj, bi2, bj2)
